# Initial kernel scaffold; baseline (speedup 1.0000x reference)
import jax
import jax.numpy as jnp
from jax.experimental import pallas as pl


def kernel(x, edge_index, params):
    raise NotImplementedError("write your pallas kernel here")


# speedup vs baseline: 5.8378x; 5.8378x over previous
import functools

import jax
import jax.numpy as jnp
from jax import lax
from jax.experimental import pallas as pl
from jax.experimental.pallas import tpu as pltpu
from jax.experimental.pallas import tpu_sc as plsc

N = 10000
E = 320000
D = 128
K = 32

NC = 2
NS = 16
NW = NC * NS
B = 128
NB = E // B
NT = NB // NW
NX = NB - NT * NW
NTP = NT + 2
RPT = 632
RPT_LAST = N - (NS - 1) * RPT

RB = 2000
GRID = N // RB


def _maxk(h):
    u = lax.bitcast_convert_type(h, jnp.uint32)
    sign = u >> jnp.uint32(31)
    u = u ^ jnp.where(sign > 0, jnp.uint32(0xFFFFFFFF), jnp.uint32(0x80000000))
    prefix = jnp.zeros((h.shape[0], 1), jnp.uint32)
    for bit in range(31, -1, -1):
        cand = prefix | jnp.uint32(1 << bit)
        cnt = jnp.sum((u >= cand).astype(jnp.int32), axis=-1, keepdims=True)
        prefix = jnp.where(cnt >= K, cand, prefix)
    return jnp.where(u >= prefix, h, 0.0)


def _entry_body(x_ref, wi_ref, bi_ref, wl_ref, bl_ref, hpre_ref, hs_ref):
    h = jnp.maximum(
        jnp.dot(x_ref[...], wi_ref[...], preferred_element_type=jnp.float32)
        + bi_ref[...], 0.0)
    hp = jnp.dot(h, wl_ref[...], preferred_element_type=jnp.float32) + bl_ref[...]
    hpre_ref[...] = hp
    hs_ref[...] = _maxk(hp)


def _mid_body(hpre_ref, nb_ref, eps_ref, w1_ref, b1_ref, w2_ref, b2_ref,
              wl_ref, bl_ref, hpre2_ref, hs2_ref):
    neigh = nb_ref[0] + nb_ref[1]
    out = (1.0 + eps_ref[0, 0]) * hpre_ref[...] + neigh
    out = jnp.maximum(
        jnp.dot(out, w1_ref[...], preferred_element_type=jnp.float32)
        + b1_ref[...], 0.0)
    h2 = jnp.dot(out, w2_ref[...], preferred_element_type=jnp.float32) + b2_ref[...]
    hp = jnp.dot(h2, wl_ref[...], preferred_element_type=jnp.float32) + bl_ref[...]
    hpre2_ref[...] = hp
    hs2_ref[...] = _maxk(hp)


def _exit_body(hpre_ref, nb_ref, eps_ref, w1_ref, b1_ref, w2_ref, b2_ref,
               wo_ref, bo_ref, y_ref):
    neigh = nb_ref[0] + nb_ref[1]
    out = (1.0 + eps_ref[0, 0]) * hpre_ref[...] + neigh
    out = jnp.maximum(
        jnp.dot(out, w1_ref[...], preferred_element_type=jnp.float32)
        + b1_ref[...], 0.0)
    h2 = jnp.dot(out, w2_ref[...], preferred_element_type=jnp.float32) + b2_ref[...]
    y_ref[...] = jnp.dot(h2, wo_ref[...], preferred_element_type=jnp.float32) + bo_ref[...]


_XSPEC = pl.BlockSpec((RB, D), lambda i: (i, 0))
_NSPEC = pl.BlockSpec((NC, RB, D), lambda i: (0, i, 0))
_WSPEC = pl.BlockSpec((D, D), lambda i: (0, 0))
_BSPEC = pl.BlockSpec((1, D), lambda i: (0, 0))
_SSPEC = pl.BlockSpec(memory_space=pltpu.SMEM)

_SD = jax.ShapeDtypeStruct

_entry = pl.pallas_call(
    _entry_body,
    grid=(GRID,),
    in_specs=[_XSPEC, _WSPEC, _BSPEC, _WSPEC, _BSPEC],
    out_specs=[_XSPEC, _XSPEC],
    out_shape=[_SD((N, D), jnp.float32), _SD((N, D), jnp.float32)],
)

_mid = pl.pallas_call(
    _mid_body,
    grid=(GRID,),
    in_specs=[_XSPEC, _NSPEC, _SSPEC, _WSPEC, _BSPEC, _WSPEC, _BSPEC,
              _WSPEC, _BSPEC],
    out_specs=[_XSPEC, _XSPEC],
    out_shape=[_SD((N, D), jnp.float32), _SD((N, D), jnp.float32)],
)

_exit = pl.pallas_call(
    _exit_body,
    grid=(GRID,),
    in_specs=[_XSPEC, _NSPEC, _SSPEC, _WSPEC, _BSPEC, _WSPEC, _BSPEC,
              _WSPEC, _BSPEC],
    out_specs=_XSPEC,
    out_shape=_SD((N, D), jnp.float32),
)


def _agg_body(h_hbm, src_hbm, dst_hbm, zeros_hbm, out_hbm,
              bids_v, src_v, dst_v, src_cur, dst_cur, rows_v, accum_sh, sem):
    c = lax.axis_index("c")
    s = lax.axis_index("s")
    w = s * NC + c

    @pl.when(s < NS - 1)
    def _zero_main():
        pltpu.sync_copy(zeros_hbm, accum_sh.at[pl.ds(s * RPT, RPT)])

    @pl.when(s == NS - 1)
    def _zero_last():
        pltpu.sync_copy(zeros_hbm.at[pl.ds(0, RPT_LAST)],
                        accum_sh.at[pl.ds((NS - 1) * RPT, RPT_LAST)])

    def _bids(ci, carry):
        ids = w + NW * (ci * 16 + lax.iota(jnp.int32, 16))
        bids_v[pl.ds(ci * 16, 16)] = jnp.minimum(ids, NB - 1)
        return carry

    lax.fori_loop(0, NTP // 16, _bids, 0)
    pltpu.async_copy(src_hbm.at[bids_v], src_v, sem).wait()
    pltpu.async_copy(dst_hbm.at[bids_v], dst_v, sem).wait()

    plsc.subcore_barrier()

    nt = jnp.where(w < NX, NT + 1, NT)

    def _edge(t, carry):
        for j in range(B // 16):
            src_cur[pl.ds(j * 16, 16)] = src_v[t, pl.ds(j * 16, 16)]
            dst_cur[pl.ds(j * 16, 16)] = dst_v[t, pl.ds(j * 16, 16)]
        pltpu.async_copy(h_hbm.at[src_cur], rows_v, sem).wait()
        pltpu.sync_copy(rows_v, accum_sh.at[dst_cur], add=True)
        return carry

    lax.fori_loop(0, nt, _edge, 0)

    plsc.subcore_barrier()

    @pl.when(s < NS - 1)
    def _out_main():
        pltpu.sync_copy(accum_sh.at[pl.ds(s * RPT, RPT)],
                        out_hbm.at[c, pl.ds(s * RPT, RPT)])

    @pl.when(s == NS - 1)
    def _out_last():
        pltpu.sync_copy(accum_sh.at[pl.ds((NS - 1) * RPT, RPT_LAST)],
                        out_hbm.at[c, pl.ds((NS - 1) * RPT, RPT_LAST)])


_agg = pl.kernel(
    _agg_body,
    out_type=_SD((NC, N, D), jnp.float32),
    mesh=plsc.VectorSubcoreMesh(
        core_axis_name="c", subcore_axis_name="s", num_cores=NC,
        num_subcores=NS),
    scratch_types=[
        pltpu.VMEM((NTP,), jnp.int32),
        pltpu.VMEM((NTP, B), jnp.int32),
        pltpu.VMEM((NTP, B), jnp.int32),
        pltpu.VMEM((B,), jnp.int32),
        pltpu.VMEM((B,), jnp.int32),
        pltpu.VMEM((B, D), jnp.float32),
        pltpu.VMEM_SHARED((N, D), jnp.float32),
        pltpu.SemaphoreType.DMA,
    ],
)


def kernel(x, edge_index, params):
    src = edge_index[0].reshape(NB, B)
    dst = edge_index[1].reshape(NB, B)
    zeros = jnp.zeros((RPT, D), jnp.float32)
    lp0, lp1 = params['layers']

    def b2(v):
        return v.reshape(1, D)

    hpre0, hs0 = _entry(x, params['W_in'], b2(params['b_in']),
                        lp0['W_lin'], b2(lp0['b_lin']))
    nb0 = _agg(hs0, src, dst, zeros)
    hpre1, hs1 = _mid(hpre0, nb0, lp0['eps'].reshape(1, 1),
                      lp0['W1'], b2(lp0['b1']), lp0['W2'], b2(lp0['b2']),
                      lp1['W_lin'], b2(lp1['b_lin']))
    nb1 = _agg(hs1, src, dst, zeros)
    y = _exit(hpre1, nb1, lp1['eps'].reshape(1, 1),
              lp1['W1'], b2(lp1['b1']), lp1['W2'], b2(lp1['b2']),
              params['W_out'], b2(params['b_out']))
    return y


# speedup vs baseline: 6.9571x; 1.1917x over previous
import functools

import jax
import jax.numpy as jnp
from jax import lax
from jax.experimental import pallas as pl
from jax.experimental.pallas import tpu as pltpu
from jax.experimental.pallas import tpu_sc as plsc

N = 10000
E = 320000
D = 128
K = 32

NC = 2
NS = 16
NW = NC * NS
B = 128
BG = 64
NB = E // B
NT = NB // NW
NX = NB - NT * NW
NTR = 80
NACC = N + 8
RPT = 632
RPT_LAST = N - (NS - 1) * RPT

RB = 2000
GRID = N // RB


def _maxk(h):
    u = lax.bitcast_convert_type(h, jnp.uint32)
    sign = u >> jnp.uint32(31)
    u = u ^ jnp.where(sign > 0, jnp.uint32(0xFFFFFFFF), jnp.uint32(0x80000000))
    prefix = jnp.zeros((h.shape[0], 1), jnp.uint32)
    for bit in range(31, -1, -1):
        cand = prefix | jnp.uint32(1 << bit)
        cnt = jnp.sum((u >= cand).astype(jnp.int32), axis=-1, keepdims=True)
        prefix = jnp.where(cnt >= K, cand, prefix)
    return jnp.where(u >= prefix, h, 0.0)


def _entry_body(x_ref, wi_ref, bi_ref, wl_ref, bl_ref, hpre_ref, hs_ref):
    h = jnp.maximum(
        jnp.dot(x_ref[...], wi_ref[...], preferred_element_type=jnp.float32)
        + bi_ref[...], 0.0)
    hp = jnp.dot(h, wl_ref[...], preferred_element_type=jnp.float32) + bl_ref[...]
    hpre_ref[...] = hp
    hs_ref[...] = _maxk(hp)


def _mid_body(hpre_ref, nb_ref, eps_ref, w1_ref, b1_ref, w2_ref, b2_ref,
              wl_ref, bl_ref, hpre2_ref, hs2_ref):
    neigh = nb_ref[0] + nb_ref[1]
    out = (1.0 + eps_ref[0, 0]) * hpre_ref[...] + neigh
    out = jnp.maximum(
        jnp.dot(out, w1_ref[...], preferred_element_type=jnp.float32)
        + b1_ref[...], 0.0)
    h2 = jnp.dot(out, w2_ref[...], preferred_element_type=jnp.float32) + b2_ref[...]
    hp = jnp.dot(h2, wl_ref[...], preferred_element_type=jnp.float32) + bl_ref[...]
    hpre2_ref[...] = hp
    hs2_ref[...] = _maxk(hp)


def _exit_body(hpre_ref, nb_ref, eps_ref, w1_ref, b1_ref, w2_ref, b2_ref,
               wo_ref, bo_ref, y_ref):
    neigh = nb_ref[0] + nb_ref[1]
    out = (1.0 + eps_ref[0, 0]) * hpre_ref[...] + neigh
    out = jnp.maximum(
        jnp.dot(out, w1_ref[...], preferred_element_type=jnp.float32)
        + b1_ref[...], 0.0)
    h2 = jnp.dot(out, w2_ref[...], preferred_element_type=jnp.float32) + b2_ref[...]
    y_ref[...] = jnp.dot(h2, wo_ref[...], preferred_element_type=jnp.float32) + bo_ref[...]


_XSPEC = pl.BlockSpec((RB, D), lambda i: (i, 0))
_NSPEC = pl.BlockSpec((NC, RB, D), lambda i: (0, i, 0))
_WSPEC = pl.BlockSpec((D, D), lambda i: (0, 0))
_BSPEC = pl.BlockSpec((1, D), lambda i: (0, 0))
_SSPEC = pl.BlockSpec(memory_space=pltpu.SMEM)

_SD = jax.ShapeDtypeStruct

_entry = pl.pallas_call(
    _entry_body,
    grid=(GRID,),
    in_specs=[_XSPEC, _WSPEC, _BSPEC, _WSPEC, _BSPEC],
    out_specs=[_XSPEC, _XSPEC],
    out_shape=[_SD((N, D), jnp.float32), _SD((N, D), jnp.float32)],
)

_mid = pl.pallas_call(
    _mid_body,
    grid=(GRID,),
    in_specs=[_XSPEC, _NSPEC, _SSPEC, _WSPEC, _BSPEC, _WSPEC, _BSPEC,
              _WSPEC, _BSPEC],
    out_specs=[_XSPEC, _XSPEC],
    out_shape=[_SD((N, D), jnp.float32), _SD((N, D), jnp.float32)],
)

_exit = pl.pallas_call(
    _exit_body,
    grid=(GRID,),
    in_specs=[_XSPEC, _NSPEC, _SSPEC, _WSPEC, _BSPEC, _WSPEC, _BSPEC,
              _WSPEC, _BSPEC],
    out_specs=_XSPEC,
    out_shape=_SD((N, D), jnp.float32),
)


def _agg_body(h_hbm, src_hbm, dst_hbm, zeros_hbm, out_hbm,
              bids_v, src_v, dst_v, dst_cur0, dst_cur1, rows0_v, rows1_v,
              accum_sh, sem0, sem1):
    c = lax.axis_index("c")
    s = lax.axis_index("s")
    w = s * NC + c

    @pl.when(s < NS - 1)
    def _zero_main():
        pltpu.sync_copy(zeros_hbm, accum_sh.at[pl.ds(s * RPT, RPT)])

    @pl.when(s == NS - 1)
    def _zero_last():
        pltpu.sync_copy(zeros_hbm.at[pl.ds(0, RPT_LAST)],
                        accum_sh.at[pl.ds((NS - 1) * RPT, RPT_LAST)])

    def _bids(ci, carry):
        ids = w + NW * (ci * 16 + lax.iota(jnp.int32, 16))
        bids_v[pl.ds(ci * 16, 16)] = jnp.minimum(ids, NB - 1)
        return carry

    lax.fori_loop(0, NTR // 16, _bids, 0)
    pltpu.async_copy(src_hbm.at[bids_v], src_v, sem0).wait()
    pltpu.async_copy(dst_hbm.at[bids_v], dst_v, sem0).wait()

    nt = jnp.where(w < NX, NT + 1, NT)

    def _pad(t, carry):
        for j in range(B // 16):
            dst_v[t, pl.ds(j * 16, 16)] = jnp.full((16,), N, jnp.int32)
        return carry

    lax.fori_loop(nt, NTR, _pad, 0)

    plsc.subcore_barrier()

    pltpu.async_copy(h_hbm.at[src_v.at[0, pl.ds(0, BG)]], rows0_v, sem0)

    def _row(t, carry):
        pltpu.async_copy(h_hbm.at[src_v.at[t, pl.ds(BG, BG)]], rows1_v, sem1)
        for j in range(BG // 16):
            dst_cur0[pl.ds(j * 16, 16)] = dst_v[t, pl.ds(j * 16, 16)]
        pltpu.make_async_copy(
            h_hbm.at[src_v.at[t, pl.ds(0, BG)]], rows0_v, sem0).wait()
        pltpu.sync_copy(rows0_v, accum_sh.at[dst_cur0], add=True)

        @pl.when(t + 1 < NTR)
        def _next():
            pltpu.async_copy(
                h_hbm.at[src_v.at[t + 1, pl.ds(0, BG)]], rows0_v, sem0)

        for j in range(BG // 16):
            dst_cur1[pl.ds(j * 16, 16)] = dst_v[t, pl.ds(BG + j * 16, 16)]
        pltpu.make_async_copy(
            h_hbm.at[src_v.at[t, pl.ds(BG, BG)]], rows1_v, sem1).wait()
        pltpu.sync_copy(rows1_v, accum_sh.at[dst_cur1], add=True)
        return carry

    lax.fori_loop(0, NTR, _row, 0)

    plsc.subcore_barrier()

    @pl.when(s < NS - 1)
    def _out_main():
        pltpu.sync_copy(accum_sh.at[pl.ds(s * RPT, RPT)],
                        out_hbm.at[c, pl.ds(s * RPT, RPT)])

    @pl.when(s == NS - 1)
    def _out_last():
        pltpu.sync_copy(accum_sh.at[pl.ds((NS - 1) * RPT, RPT_LAST)],
                        out_hbm.at[c, pl.ds((NS - 1) * RPT, RPT_LAST)])


_agg = pl.kernel(
    _agg_body,
    out_type=_SD((NC, N, D), jnp.float32),
    mesh=plsc.VectorSubcoreMesh(
        core_axis_name="c", subcore_axis_name="s", num_cores=NC,
        num_subcores=NS),
    scratch_types=[
        pltpu.VMEM((NTR,), jnp.int32),
        pltpu.VMEM((NTR, B), jnp.int32),
        pltpu.VMEM((NTR, B), jnp.int32),
        pltpu.VMEM((BG,), jnp.int32),
        pltpu.VMEM((BG,), jnp.int32),
        pltpu.VMEM((BG, D), jnp.float32),
        pltpu.VMEM((BG, D), jnp.float32),
        pltpu.VMEM_SHARED((NACC, D), jnp.float32),
        pltpu.SemaphoreType.DMA,
        pltpu.SemaphoreType.DMA,
    ],
)


def kernel(x, edge_index, params):
    src = edge_index[0].reshape(NB, B)
    dst = edge_index[1].reshape(NB, B)
    zeros = jnp.zeros((RPT, D), jnp.float32)
    lp0, lp1 = params['layers']

    def b2(v):
        return v.reshape(1, D)

    hpre0, hs0 = _entry(x, params['W_in'], b2(params['b_in']),
                        lp0['W_lin'], b2(lp0['b_lin']))
    nb0 = _agg(hs0, src, dst, zeros)
    hpre1, hs1 = _mid(hpre0, nb0, lp0['eps'].reshape(1, 1),
                      lp0['W1'], b2(lp0['b1']), lp0['W2'], b2(lp0['b2']),
                      lp1['W_lin'], b2(lp1['b_lin']))
    nb1 = _agg(hs1, src, dst, zeros)
    y = _exit(hpre1, nb1, lp1['eps'].reshape(1, 1),
              lp1['W1'], b2(lp1['b1']), lp1['W2'], b2(lp1['b2']),
              params['W_out'], b2(params['b_out']))
    return y


# speedup vs baseline: 8.6089x; 1.2374x over previous
import functools

import jax
import jax.numpy as jnp
from jax import lax
from jax.experimental import pallas as pl
from jax.experimental.pallas import tpu as pltpu
from jax.experimental.pallas import tpu_sc as plsc

N = 10000
E = 320000
D = 128
K = 32

NC = 2
NS = 16
NW = NC * NS
B = 128
BG = 64
NB = E // B
NT = NB // NW
NX = NB - NT * NW
NTR = 80
NACC = N + 8
RPT = 632
RPT_LAST = N - (NS - 1) * RPT

RB = 2000
GRID = N // RB


def _maxk(h):
    u = lax.bitcast_convert_type(h, jnp.uint32)
    sign = u >> jnp.uint32(31)
    u = u ^ jnp.where(sign > 0, jnp.uint32(0xFFFFFFFF), jnp.uint32(0x80000000))
    ones = jnp.ones((D, 1), jnp.float32)
    prefix = jnp.zeros((h.shape[0], 1), jnp.uint32)
    for bit in range(31, -1, -1):
        cand = prefix | jnp.uint32(1 << bit)
        mask = (u >= cand).astype(jnp.float32)
        cnt = jnp.dot(mask, ones, preferred_element_type=jnp.float32)
        prefix = jnp.where(cnt >= float(K), cand, prefix)
    return jnp.where(u >= prefix, h, 0.0)


def _entry_body(x_ref, wi_ref, bi_ref, wl_ref, bl_ref, hpre_ref, hs_ref):
    h = jnp.maximum(
        jnp.dot(x_ref[...], wi_ref[...], preferred_element_type=jnp.float32)
        + bi_ref[...], 0.0)
    hp = jnp.dot(h, wl_ref[...], preferred_element_type=jnp.float32) + bl_ref[...]
    hpre_ref[...] = hp
    hs_ref[...] = _maxk(hp)


def _mid_body(hpre_ref, nb_ref, eps_ref, w1_ref, b1_ref, w2_ref, b2_ref,
              wl_ref, bl_ref, hpre2_ref, hs2_ref):
    neigh = nb_ref[0] + nb_ref[1]
    out = (1.0 + eps_ref[0, 0]) * hpre_ref[...] + neigh
    out = jnp.maximum(
        jnp.dot(out, w1_ref[...], preferred_element_type=jnp.float32)
        + b1_ref[...], 0.0)
    h2 = jnp.dot(out, w2_ref[...], preferred_element_type=jnp.float32) + b2_ref[...]
    hp = jnp.dot(h2, wl_ref[...], preferred_element_type=jnp.float32) + bl_ref[...]
    hpre2_ref[...] = hp
    hs2_ref[...] = _maxk(hp)


def _exit_body(hpre_ref, nb_ref, eps_ref, w1_ref, b1_ref, w2_ref, b2_ref,
               wo_ref, bo_ref, y_ref):
    neigh = nb_ref[0] + nb_ref[1]
    out = (1.0 + eps_ref[0, 0]) * hpre_ref[...] + neigh
    out = jnp.maximum(
        jnp.dot(out, w1_ref[...], preferred_element_type=jnp.float32)
        + b1_ref[...], 0.0)
    h2 = jnp.dot(out, w2_ref[...], preferred_element_type=jnp.float32) + b2_ref[...]
    y_ref[...] = jnp.dot(h2, wo_ref[...], preferred_element_type=jnp.float32) + bo_ref[...]


_XSPEC = pl.BlockSpec((RB, D), lambda i: (i, 0))
_NSPEC = pl.BlockSpec((NC, RB, D), lambda i: (0, i, 0))
_WSPEC = pl.BlockSpec((D, D), lambda i: (0, 0))
_BSPEC = pl.BlockSpec((1, D), lambda i: (0, 0))
_SSPEC = pl.BlockSpec(memory_space=pltpu.SMEM)

_SD = jax.ShapeDtypeStruct

_entry = pl.pallas_call(
    _entry_body,
    grid=(GRID,),
    in_specs=[_XSPEC, _WSPEC, _BSPEC, _WSPEC, _BSPEC],
    out_specs=[_XSPEC, _XSPEC],
    out_shape=[_SD((N, D), jnp.float32), _SD((N, D), jnp.float32)],
)

_mid = pl.pallas_call(
    _mid_body,
    grid=(GRID,),
    in_specs=[_XSPEC, _NSPEC, _SSPEC, _WSPEC, _BSPEC, _WSPEC, _BSPEC,
              _WSPEC, _BSPEC],
    out_specs=[_XSPEC, _XSPEC],
    out_shape=[_SD((N, D), jnp.float32), _SD((N, D), jnp.float32)],
)

_exit = pl.pallas_call(
    _exit_body,
    grid=(GRID,),
    in_specs=[_XSPEC, _NSPEC, _SSPEC, _WSPEC, _BSPEC, _WSPEC, _BSPEC,
              _WSPEC, _BSPEC],
    out_specs=_XSPEC,
    out_shape=_SD((N, D), jnp.float32),
)


def _agg_body(h_hbm, src_hbm, dst_hbm, zeros_hbm, out_hbm,
              bids_v, src_v, dst_v, dst_cur0, dst_cur1, rows0_v, rows1_v,
              accum_sh, sem0, sem1):
    c = lax.axis_index("c")
    s = lax.axis_index("s")
    w = s * NC + c

    @pl.when(s < NS - 1)
    def _zero_main():
        pltpu.sync_copy(zeros_hbm, accum_sh.at[pl.ds(s * RPT, RPT)])

    @pl.when(s == NS - 1)
    def _zero_last():
        pltpu.sync_copy(zeros_hbm.at[pl.ds(0, RPT_LAST)],
                        accum_sh.at[pl.ds((NS - 1) * RPT, RPT_LAST)])

    def _bids(ci, carry):
        ids = w + NW * (ci * 16 + lax.iota(jnp.int32, 16))
        bids_v[pl.ds(ci * 16, 16)] = jnp.minimum(ids, NB - 1)
        return carry

    lax.fori_loop(0, NTR // 16, _bids, 0)
    pltpu.async_copy(src_hbm.at[bids_v], src_v, sem0).wait()
    pltpu.async_copy(dst_hbm.at[bids_v], dst_v, sem0).wait()

    nt = jnp.where(w < NX, NT + 1, NT)

    def _pad(t, carry):
        for j in range(B // 16):
            dst_v[t, pl.ds(j * 16, 16)] = jnp.full((16,), N, jnp.int32)
        return carry

    lax.fori_loop(nt, NTR, _pad, 0)

    plsc.subcore_barrier()

    pltpu.async_copy(h_hbm.at[src_v.at[0, pl.ds(0, BG)]], rows0_v, sem0)

    def _row(t, carry):
        pltpu.async_copy(h_hbm.at[src_v.at[t, pl.ds(BG, BG)]], rows1_v, sem1)
        for j in range(BG // 16):
            dst_cur0[pl.ds(j * 16, 16)] = dst_v[t, pl.ds(j * 16, 16)]
        pltpu.make_async_copy(
            h_hbm.at[src_v.at[t, pl.ds(0, BG)]], rows0_v, sem0).wait()
        pltpu.sync_copy(rows0_v, accum_sh.at[dst_cur0], add=True)

        @pl.when(t + 1 < NTR)
        def _next():
            pltpu.async_copy(
                h_hbm.at[src_v.at[t + 1, pl.ds(0, BG)]], rows0_v, sem0)

        for j in range(BG // 16):
            dst_cur1[pl.ds(j * 16, 16)] = dst_v[t, pl.ds(BG + j * 16, 16)]
        pltpu.make_async_copy(
            h_hbm.at[src_v.at[t, pl.ds(BG, BG)]], rows1_v, sem1).wait()
        pltpu.sync_copy(rows1_v, accum_sh.at[dst_cur1], add=True)
        return carry

    lax.fori_loop(0, NTR, _row, 0)

    plsc.subcore_barrier()

    @pl.when(s < NS - 1)
    def _out_main():
        pltpu.sync_copy(accum_sh.at[pl.ds(s * RPT, RPT)],
                        out_hbm.at[c, pl.ds(s * RPT, RPT)])

    @pl.when(s == NS - 1)
    def _out_last():
        pltpu.sync_copy(accum_sh.at[pl.ds((NS - 1) * RPT, RPT_LAST)],
                        out_hbm.at[c, pl.ds((NS - 1) * RPT, RPT_LAST)])


_agg = pl.kernel(
    _agg_body,
    out_type=_SD((NC, N, D), jnp.float32),
    mesh=plsc.VectorSubcoreMesh(
        core_axis_name="c", subcore_axis_name="s", num_cores=NC,
        num_subcores=NS),
    scratch_types=[
        pltpu.VMEM((NTR,), jnp.int32),
        pltpu.VMEM((NTR, B), jnp.int32),
        pltpu.VMEM((NTR, B), jnp.int32),
        pltpu.VMEM((BG,), jnp.int32),
        pltpu.VMEM((BG,), jnp.int32),
        pltpu.VMEM((BG, D), jnp.float32),
        pltpu.VMEM((BG, D), jnp.float32),
        pltpu.VMEM_SHARED((NACC, D), jnp.float32),
        pltpu.SemaphoreType.DMA,
        pltpu.SemaphoreType.DMA,
    ],
)


def kernel(x, edge_index, params):
    src = edge_index[0].reshape(NB, B)
    dst = edge_index[1].reshape(NB, B)
    zeros = jnp.zeros((RPT, D), jnp.float32)
    lp0, lp1 = params['layers']

    def b2(v):
        return v.reshape(1, D)

    hpre0, hs0 = _entry(x, params['W_in'], b2(params['b_in']),
                        lp0['W_lin'], b2(lp0['b_lin']))
    nb0 = _agg(hs0, src, dst, zeros)
    hpre1, hs1 = _mid(hpre0, nb0, lp0['eps'].reshape(1, 1),
                      lp0['W1'], b2(lp0['b1']), lp0['W2'], b2(lp0['b2']),
                      lp1['W_lin'], b2(lp1['b_lin']))
    nb1 = _agg(hs1, src, dst, zeros)
    y = _exit(hpre1, nb1, lp1['eps'].reshape(1, 1),
              lp1['W1'], b2(lp1['b1']), lp1['W2'], b2(lp1['b2']),
              params['W_out'], b2(params['b_out']))
    return y
